# baseline (speedup 1.0000x reference)
import jax
import jax.numpy as jnp
from jax.experimental import pallas as pl
from jax.experimental.pallas import tpu as pltpu


_NUM_CHUNKS = 32


def _dma_body(in_hbm, out_hbm, scratch, bulk_sem, tile_sem):
    n_rows, n_cols = in_hbm.shape
    chunk = n_rows // _NUM_CHUNKS

    copies = []
    for k in range(_NUM_CHUNKS):
        sl = (pl.ds(k * chunk, chunk), slice(None))
        cp = pltpu.make_async_copy(in_hbm.at[sl], out_hbm.at[sl], bulk_sem.at[k])
        cp.start()
        copies.append(cp)

    top = (pl.ds(0, 8), pl.ds(n_cols - 128, 128))
    bot = (pl.ds(n_rows - 8, 8), pl.ds(0, 128))
    ld_top = pltpu.make_async_copy(in_hbm.at[top], scratch.at[0], tile_sem)
    ld_top.start()
    ld_top.wait()
    ld_bot = pltpu.make_async_copy(in_hbm.at[bot], scratch.at[1], tile_sem)
    ld_bot.start()
    ld_bot.wait()

    r = jax.lax.broadcasted_iota(jnp.int32, (8, 128), 0)
    c = jax.lax.broadcasted_iota(jnp.int32, (8, 128), 1)
    scratch[0] = jnp.where((r == 0) & (c == 126), jnp.float32(1.0), scratch[0])
    scratch[1] = jnp.where((r == 7) & (c == 1), jnp.float32(5.0), scratch[1])

    for cp in copies:
        cp.wait()
    st_top = pltpu.make_async_copy(scratch.at[0], out_hbm.at[top], tile_sem)
    st_top.start()
    st_top.wait()
    st_bot = pltpu.make_async_copy(scratch.at[1], out_hbm.at[bot], tile_sem)
    st_bot.start()
    st_bot.wait()


@jax.jit
def kernel(x):
    return pl.pallas_call(
        _dma_body,
        in_specs=[pl.BlockSpec(memory_space=pl.ANY)],
        out_specs=pl.BlockSpec(memory_space=pl.ANY),
        out_shape=jax.ShapeDtypeStruct(x.shape, x.dtype),
        scratch_shapes=[
            pltpu.VMEM((2, 8, 128), jnp.float32),
            pltpu.SemaphoreType.DMA((_NUM_CHUNKS,)),
            pltpu.SemaphoreType.DMA,
        ],
    )(x)


# speedup vs baseline: 48.4876x; 48.4876x over previous
import functools

import jax
import jax.numpy as jnp
from jax.experimental import pallas as pl

_BLOCK_ROWS = 256


def _patch_tile(out_ref, rows, cols, row, col, value):
    tile = out_ref[rows, cols]
    r = jax.lax.broadcasted_iota(jnp.int32, tile.shape, 0)
    c = jax.lax.broadcasted_iota(jnp.int32, tile.shape, 1)
    mask = (r == row) & (c == col)
    out_ref[rows, cols] = jnp.where(mask, jnp.float32(value), tile)


def _copy_patch_body(in_ref, out_ref, *, n_cols, num_blocks, block_rows):
    out_ref[...] = in_ref[...]
    i = pl.program_id(0)

    @pl.when(i == 0)
    def _():
        _patch_tile(out_ref, pl.ds(0, 8), pl.ds(n_cols - 128, 128), 0, 126, 1.0)

    @pl.when(i == num_blocks - 1)
    def _():
        _patch_tile(out_ref, pl.ds(block_rows - 8, 8), pl.ds(0, 128), 7, 1, 5.0)


@jax.jit
def kernel(x):
    n_rows, n_cols = x.shape
    block_rows = _BLOCK_ROWS
    num_blocks = n_rows // block_rows
    body = functools.partial(
        _copy_patch_body,
        n_cols=n_cols,
        num_blocks=num_blocks,
        block_rows=block_rows,
    )
    return pl.pallas_call(
        body,
        grid=(num_blocks,),
        in_specs=[pl.BlockSpec((block_rows, n_cols), lambda i: (i, 0))],
        out_specs=pl.BlockSpec((block_rows, n_cols), lambda i: (i, 0)),
        out_shape=jax.ShapeDtypeStruct(x.shape, x.dtype),
    )(x)
